# baseline (speedup 1.0000x reference)
import functools

import jax
import jax.numpy as jnp
from jax import lax
from jax.experimental import pallas as pl
from jax.experimental.pallas import tpu as pltpu
from jax.experimental.pallas import tpu_sc as plsc

D = 64
B = 16384
V = 100000
NC, NS, L = 2, 16, 16
NW = NC * NS
BPW = B // NW
CHUNK = 128
NCH = BPW // CHUNK
NG = BPW // L

CB = 50
NCK = V // 8 // CB
DROUNDS = -(-NCK // NW)

_mesh = plsc.VectorSubcoreMesh(
    core_axis_name="c", subcore_axis_name="s", num_cores=NC, num_subcores=NS)

_sc_params = pltpu.CompilerParams(
    needs_layout_passes=False, disable_bounds_checks=True)


@functools.partial(
    pl.kernel,
    out_type=jax.ShapeDtypeStruct((V // 2, 2 * D), jnp.float32),
    mesh=_mesh,
    compiler_params=_sc_params,
    scratch_types=[pltpu.VMEM((CB, 8, D), jnp.float32),
                   pltpu.VMEM((CB * 4, 2 * D), jnp.float32)],
)
def _sc_depad(cent_hbm, lin_hbm, buf, obuf):
    wid = lax.axis_index("s") * NC + lax.axis_index("c")

    def chunk_body(i, _):
        j = wid + i * NW

        @pl.when(j < NCK)
        def _():
            pltpu.sync_copy(cent_hbm.at[pl.ds(j * CB, CB)], buf)
            halves = buf.reshape(CB * 4, 2, D)

            @plsc.parallel_loop(0, CB * 4, 1, unroll=4)
            def _copy(k):
                for p in range(2):
                    for c in range(D // L):
                        obuf[k, pl.ds(p * D + c * L, L)] = (
                            halves[k, p, pl.ds(c * L, L)])

            pltpu.sync_copy(obuf, lin_hbm.at[pl.ds(j * CB * 4, CB * 4)])

        return 0

    lax.fori_loop(0, DROUNDS, chunk_body, 0)


@functools.partial(
    pl.kernel,
    out_type=jax.ShapeDtypeStruct((NW, L), jnp.float32),
    mesh=_mesh,
    compiler_params=_sc_params,
    scratch_types=[
        pltpu.VMEM((NCH, CHUNK), jnp.int32),
        pltpu.VMEM((NCH, CHUNK), jnp.int32),
        pltpu.VMEM((BPW, D), jnp.float32),
        pltpu.VMEM((BPW // 2, 2 * D), jnp.float32),
        pltpu.VMEM((L,), jnp.float32),
        pltpu.SemaphoreType.DMA,
        pltpu.SemaphoreType.DMA,
    ],
)
def _sc_partials(feat_hbm, lab_hbm, lin_hbm, out_hbm,
                 idx_v, gidx_v, feat_v, rows_v, acc_v, gsem, fsem):
    wid = lax.axis_index("s") * NC + lax.axis_index("c")
    pltpu.sync_copy(lab_hbm.at[pl.ds(wid * NCH, NCH)], idx_v)
    fcp = pltpu.async_copy(
        feat_hbm.at[pl.ds(wid * BPW, BPW)], feat_v, fsem)
    for k in range(NCH * CHUNK // L):
        r, c0 = k // (CHUNK // L), (k % (CHUNK // L)) * L
        gidx_v[r, pl.ds(c0, L)] = idx_v[r, pl.ds(c0, L)] >> jnp.int32(1)
    fcp.wait()
    acc = jnp.zeros((L,), jnp.float32)
    for h in range(2):
        gcps = [
            pltpu.async_copy(lin_hbm.at[gidx_v.at[2 * h + j]],
                             rows_v.at[pl.ds(j * CHUNK, CHUNK)], gsem)
            for j in range(2)
        ]
        for g in gcps:
            g.wait()

        def group_body(g, acc):
            labs = idx_v[g >> jnp.int32(3), pl.ds((g & jnp.int32(7)) * L, L)]
            for i in range(L):
                off = (labs[i] & jnp.int32(1)) * jnp.int32(D)
                r = g * L + i
                rl = r - h * (BPW // 2)
                dist = jnp.zeros((L,), jnp.float32)
                for c in range(D // L):
                    f = feat_v[r, pl.ds(c * L, L)]
                    t = rows_v[rl, pl.ds(off + c * L, L)]
                    dlt = f - t
                    dist = dist + dlt * dlt
                acc = acc + dist
            return acc

        acc = lax.fori_loop(h * (NG // 2), (h + 1) * (NG // 2),
                            group_body, acc)
    acc_v[...] = acc
    pltpu.sync_copy(acc_v, out_hbm.at[wid])


def _tc_mean_body(p_ref, o_ref):
    o_ref[0, 0] = jnp.sum(p_ref[...]) * (1.0 / B)


_tc_mean = pl.pallas_call(
    _tc_mean_body,
    out_shape=jax.ShapeDtypeStruct((1, 1), jnp.float32),
    out_specs=pl.BlockSpec(memory_space=pltpu.SMEM),
)


def kernel(features, labels, centers):
    lab2 = labels.astype(jnp.int32).reshape(128, 128)
    cent3 = centers.reshape(V // 8, 8, D)
    lin = _sc_depad(cent3)
    partials = _sc_partials(features, lab2, lin)
    return _tc_mean(partials)[0, 0]


# speedup vs baseline: 1.0510x; 1.0510x over previous
import functools

import jax
import jax.numpy as jnp
from jax import lax
from jax.experimental import pallas as pl
from jax.experimental.pallas import tpu as pltpu
from jax.experimental.pallas import tpu_sc as plsc

D = 64
B = 16384
V = 100000
NC, NS, L = 2, 16, 16
NW = NC * NS
BPW = B // NW
CHUNK = 128
NCH = BPW // CHUNK
NG = BPW // L

CB = 50
NCK = V // 8 // CB
DROUNDS = -(-NCK // NW)

_mesh = plsc.VectorSubcoreMesh(
    core_axis_name="c", subcore_axis_name="s", num_cores=NC, num_subcores=NS)

_sc_params = pltpu.CompilerParams(
    needs_layout_passes=False, disable_bounds_checks=True)


@functools.partial(
    pl.kernel,
    out_type=jax.ShapeDtypeStruct((V // 2, 2 * D), jnp.float32),
    mesh=_mesh,
    compiler_params=_sc_params,
    scratch_types=[pltpu.VMEM((CB, 8, D), jnp.float32),
                   pltpu.VMEM((CB * 4, 2 * D), jnp.float32),
                   pltpu.SemaphoreType.DMA,
                   pltpu.SemaphoreType.DMA],
)
def _sc_depad(cent_hbm, lin_hbm, buf, obuf, isem, osem):
    wid = lax.axis_index("s") * NC + lax.axis_index("c")

    def in_copy(j):
        return pltpu.make_async_copy(
            cent_hbm.at[pl.ds(j * CB, CB)], buf, isem)

    def out_copy(j):
        return pltpu.make_async_copy(
            obuf, lin_hbm.at[pl.ds(j * CB * 4, CB * 4)], osem)

    @pl.when(wid < NCK)
    def _prime():
        in_copy(wid).start()

    def chunk_body(i, _):
        j = wid + i * NW

        @pl.when(j < NCK)
        def _():
            in_copy(j).wait()
            halves = buf.reshape(CB * 4, 2, D)

            @pl.when(i > 0)
            def _drain():
                out_copy(j - NW).wait()

            @plsc.parallel_loop(0, CB * 4, 1, unroll=4)
            def _copy(k):
                for p in range(2):
                    for c in range(D // L):
                        obuf[k, pl.ds(p * D + c * L, L)] = (
                            halves[k, p, pl.ds(c * L, L)])

            out_copy(j).start()

            @pl.when(j + NW < NCK)
            def _prefetch():
                in_copy(j + NW).start()

        return 0

    lax.fori_loop(0, DROUNDS, chunk_body, 0)

    @pl.when(wid < NCK)
    def _final_drain():
        last = wid + ((NCK - 1 - wid) // NW) * NW
        out_copy(last).wait()


@functools.partial(
    pl.kernel,
    out_type=jax.ShapeDtypeStruct((NW, L), jnp.float32),
    mesh=_mesh,
    compiler_params=_sc_params,
    scratch_types=[
        pltpu.VMEM((NCH, CHUNK), jnp.int32),
        pltpu.VMEM((NCH, CHUNK), jnp.int32),
        pltpu.VMEM((BPW, D), jnp.float32),
        pltpu.VMEM((BPW // 2, 2 * D), jnp.float32),
        pltpu.VMEM((L,), jnp.float32),
        pltpu.SemaphoreType.DMA,
        pltpu.SemaphoreType.DMA,
    ],
)
def _sc_partials(feat_hbm, lab_hbm, lin_hbm, out_hbm,
                 idx_v, gidx_v, feat_v, rows_v, acc_v, gsem, fsem):
    wid = lax.axis_index("s") * NC + lax.axis_index("c")
    pltpu.sync_copy(lab_hbm.at[pl.ds(wid * NCH, NCH)], idx_v)
    fcp = pltpu.async_copy(
        feat_hbm.at[pl.ds(wid * BPW, BPW)], feat_v, fsem)
    for k in range(NCH * CHUNK // L):
        r, c0 = k // (CHUNK // L), (k % (CHUNK // L)) * L
        gidx_v[r, pl.ds(c0, L)] = idx_v[r, pl.ds(c0, L)] >> jnp.int32(1)
    fcp.wait()
    acc = jnp.zeros((L,), jnp.float32)
    for h in range(2):
        gcps = [
            pltpu.async_copy(lin_hbm.at[gidx_v.at[2 * h + j]],
                             rows_v.at[pl.ds(j * CHUNK, CHUNK)], gsem)
            for j in range(2)
        ]
        for g in gcps:
            g.wait()

        def group_body(g, acc):
            labs = idx_v[g >> jnp.int32(3), pl.ds((g & jnp.int32(7)) * L, L)]
            for i in range(L):
                off = (labs[i] & jnp.int32(1)) * jnp.int32(D)
                r = g * L + i
                rl = r - h * (BPW // 2)
                dist = jnp.zeros((L,), jnp.float32)
                for c in range(D // L):
                    f = feat_v[r, pl.ds(c * L, L)]
                    t = rows_v[rl, pl.ds(off + c * L, L)]
                    dlt = f - t
                    dist = dist + dlt * dlt
                acc = acc + dist
            return acc

        acc = lax.fori_loop(h * (NG // 2), (h + 1) * (NG // 2),
                            group_body, acc)
    acc_v[...] = acc
    pltpu.sync_copy(acc_v, out_hbm.at[wid])


def _tc_mean_body(p_ref, o_ref):
    o_ref[0, 0] = jnp.sum(p_ref[...]) * (1.0 / B)


_tc_mean = pl.pallas_call(
    _tc_mean_body,
    out_shape=jax.ShapeDtypeStruct((1, 1), jnp.float32),
    out_specs=pl.BlockSpec(memory_space=pltpu.SMEM),
)


def kernel(features, labels, centers):
    lab2 = labels.astype(jnp.int32).reshape(128, 128)
    cent3 = centers.reshape(V // 8, 8, D)
    lin = _sc_depad(cent3)
    partials = _sc_partials(features, lab2, lin)
    return _tc_mean(partials)[0, 0]


# speedup vs baseline: 1.0746x; 1.0224x over previous
import functools

import jax
import jax.numpy as jnp
from jax import lax
from jax.experimental import pallas as pl
from jax.experimental.pallas import tpu as pltpu
from jax.experimental.pallas import tpu_sc as plsc

D = 64
B = 16384
NC, NS, L = 2, 16, 16
NW = NC * NS
BPW = B // NW
CHUNK = 128
NCH = BPW // CHUNK

_mesh = plsc.VectorSubcoreMesh(
    core_axis_name="c", subcore_axis_name="s", num_cores=NC, num_subcores=NS)


@functools.partial(
    pl.kernel,
    out_type=jax.ShapeDtypeStruct((NW, L), jnp.float32),
    mesh=_mesh,
    compiler_params=pltpu.CompilerParams(use_tc_tiling_on_sc=False),
    scratch_types=[
        pltpu.VMEM((NCH, CHUNK), jnp.int32),
        pltpu.VMEM((BPW, D), jnp.float32),
        pltpu.VMEM((BPW, D), jnp.float32),
        pltpu.VMEM((L,), jnp.float32),
        pltpu.SemaphoreType.DMA,
        pltpu.SemaphoreType.DMA,
    ],
)
def _sc_partials(feat_hbm, lab_hbm, cent_hbm, out_hbm,
                 idx_v, feat_v, rows_v, acc_v, gsem, fsem):
    wid = lax.axis_index("s") * NC + lax.axis_index("c")
    base = wid * BPW
    pltpu.sync_copy(lab_hbm.at[wid], idx_v)
    fcp = pltpu.async_copy(feat_hbm.at[pl.ds(base, BPW)], feat_v, fsem)
    gcps = [
        pltpu.async_copy(cent_hbm.at[idx_v.at[j]],
                         rows_v.at[pl.ds(j * CHUNK, CHUNK)], gsem)
        for j in range(NCH)
    ]
    fcp.wait()
    for g in gcps:
        g.wait()

    def body(r, accs):
        out = []
        for c in range(D // L):
            f = feat_v[r, pl.ds(c * L, L)]
            g = rows_v[r, pl.ds(c * L, L)]
            dlt = f - g
            out.append(accs[c] + dlt * dlt)
        return tuple(out)

    zero = jnp.zeros((L,), jnp.float32)
    accs = lax.fori_loop(0, BPW, body, (zero,) * (D // L))
    acc_v[...] = (accs[0] + accs[1]) + (accs[2] + accs[3])
    pltpu.sync_copy(acc_v, out_hbm.at[wid])


def _tc_mean_body(p_ref, o_ref):
    o_ref[0, 0] = jnp.sum(p_ref[...]) * (1.0 / B)


_tc_mean = pl.pallas_call(
    _tc_mean_body,
    out_shape=jax.ShapeDtypeStruct((1, 1), jnp.float32),
    out_specs=pl.BlockSpec(memory_space=pltpu.SMEM),
)


def kernel(features, labels, centers):
    lab = labels.astype(jnp.int32).reshape(NW, NCH, CHUNK)
    partials = _sc_partials(features, lab, centers)
    return _tc_mean(partials)[0, 0]
